# baseline (speedup 1.0000x reference)
import functools

import jax
import jax.numpy as jnp
from jax import lax
from jax.experimental import pallas as pl
from jax.experimental.pallas import tpu as pltpu
from jax.experimental.pallas import tpu_sc as plsc

N = 10000
E = 160000
R = 8
D = 128

NC = 2
NS = 16
NW = NC * NS

EPW = 5120
BE = 64
NBK = EPW // BE
NBC = EPW // 128
EPAD = NW * EPW - E
NR_PAD = 81920
NPAD = 10240
RPT = NPAD // NS

_mesh = plsc.VectorSubcoreMesh(core_axis_name="c", subcore_axis_name="s")


@functools.partial(
    pl.kernel,
    out_type=jax.ShapeDtypeStruct((NW, EPW), jnp.float32),
    mesh=_mesh,
    compiler_params=pltpu.CompilerParams(needs_layout_passes=False),
    scratch_types=[
        pltpu.VMEM((NBC, 128), jnp.int32),
        pltpu.VMEM((NR_PAD,), jnp.float32),
        pltpu.VMEM((EPW,), jnp.float32),
        pltpu.VMEM((128,), jnp.float32),
        pltpu.VMEM_SHARED((NR_PAD,), jnp.float32),
    ],
)
def _norm_kernel(key_hbm, norm_hbm, keyv, cntv, normv, onesv, cnt_sp):
    c = lax.axis_index("c")
    s = lax.axis_index("s")
    wid = s * NC + c

    zlen = NR_PAD // NS

    def _z16(i, _):
        cntv[pl.ds(i * 16, 16)] = jnp.zeros((16,), jnp.float32)
        return 0

    lax.fori_loop(0, zlen // 16, _z16, 0)
    pltpu.sync_copy(cntv.at[pl.ds(0, zlen)], cnt_sp.at[pl.ds(s * zlen, zlen)])

    def _o16(i, _):
        onesv[pl.ds(i * 16, 16)] = jnp.ones((16,), jnp.float32)
        return 0

    lax.fori_loop(0, 8, _o16, 0)
    plsc.subcore_barrier()

    def _count_row(rr, _):
        row = 2 * s + rr
        pltpu.sync_copy(key_hbm.at[row], keyv)

        def _b(b, _2):
            pltpu.sync_copy(onesv, cnt_sp.at[keyv.at[b]], add=True)
            return 0

        lax.fori_loop(0, NBC, _b, 0)
        return 0

    lax.fori_loop(0, 2, _count_row, 0)
    plsc.subcore_barrier()

    pltpu.sync_copy(cnt_sp, cntv)
    pltpu.sync_copy(key_hbm.at[wid], keyv)

    def _nb(b, _):
        for g in range(8):
            k16 = keyv[b, pl.ds(g * 16, 16)]
            c16 = plsc.load_gather(cntv, [k16])
            normv[pl.ds(b * 128 + g * 16, 16)] = 1.0 / jnp.maximum(c16, 1.0)
        return 0

    lax.fori_loop(0, NBC, _nb, 0)
    pltpu.sync_copy(normv, norm_hbm.at[wid])


@functools.partial(
    pl.kernel,
    out_type=jax.ShapeDtypeStruct((NC, NPAD, D), jnp.float32),
    mesh=_mesh,
    compiler_params=pltpu.CompilerParams(needs_layout_passes=False),
    scratch_types=[
        pltpu.VMEM((NBC, 128), jnp.int32),
        pltpu.VMEM((NBK, BE), jnp.int32),
        pltpu.VMEM((EPW,), jnp.float32),
        pltpu.VMEM((BE, D), jnp.float32),
        pltpu.VMEM((BE, D), jnp.float32),
        pltpu.VMEM((BE, D), jnp.float32),
        pltpu.VMEM_SHARED((NPAD, D), jnp.float32),
        pltpu.SemaphoreType.DMA,
        pltpu.SemaphoreType.DMA,
        pltpu.SemaphoreType.DMA,
    ],
)
def _agg_kernel(hr_hbm, idx_hbm, dst_hbm, norm_hbm, out_hbm,
                idxv, dstv, normv, rows0, rows1, rows2, acc_sp,
                gsem0, gsem1, gsem2):
    c = lax.axis_index("c")
    s = lax.axis_index("s")
    wid = s * NC + c

    pltpu.sync_copy(idx_hbm.at[wid], idxv)
    pltpu.sync_copy(dst_hbm.at[wid], dstv)
    pltpu.sync_copy(norm_hbm.at[wid], normv)

    def _zr(i, _):
        for k in range(D // 16):
            rows0[i, pl.ds(k * 16, 16)] = jnp.zeros((16,), jnp.float32)
        return 0

    lax.fori_loop(0, BE, _zr, 0)
    for t in range(RPT // BE):
        pltpu.sync_copy(rows0, acc_sp.at[pl.ds(s * RPT + t * BE, BE)])
    plsc.subcore_barrier()

    def _scale_block(rows, b):
        def _edge(j, e16):
            nsp = plsc.load_gather(normv, [e16])
            for k in range(D // 16):
                rows[j, pl.ds(k * 16, 16)] = rows[j, pl.ds(k * 16, 16)] * nsp
            return e16 + 1

        lax.fori_loop(0, BE, _edge, lax.broadcast(b * BE, (16,)))

    bufs = (rows0, rows1, rows2)
    sems = (gsem0, gsem1, gsem2)

    def _gather_start(b, par, rows, gsem):
        pltpu.async_copy(hr_hbm.at[idxv.at[b // 2, pl.ds(par * BE, BE)]],
                         rows, gsem)

    def _gather_wait(rows, gsem):
        pltpu.make_async_copy(hr_hbm.at[idxv.at[0, pl.ds(0, BE)]], rows, gsem)\
            .wait()

    for p in range(3):
        _gather_start(p, p % 2, bufs[p], sems[p])

    def _phase(i, p):
        b = 6 * i + p
        rows = bufs[p % 3]
        gsem = sems[p % 3]
        _gather_wait(rows, gsem)
        _scale_block(rows, b)
        pltpu.sync_copy(rows, acc_sp.at[dstv.at[b]], add=True)

        @pl.when(b + 3 < NBK)
        def _():
            _gather_start(b + 3, (p + 3) % 2, rows, gsem)

    def _six(i, _):
        for p in range(6):
            _phase(i, p)
        return 0

    lax.fori_loop(0, NBK // 6, _six, 0)
    for b in range(NBK - NBK % 6, NBK):
        _phase(b // 6, b % 6)
    plsc.subcore_barrier()
    pltpu.sync_copy(acc_sp.at[pl.ds(s * RPT, RPT)],
                    out_hbm.at[c, pl.ds(s * RPT, RPT)])


_BN = 1000


def _mm_first_body(x_ref, w_ref, hr_ref, self_ref):
    o = jnp.dot(x_ref[...].astype(jnp.bfloat16), w_ref[...],
                preferred_element_type=jnp.float32)
    hr_ref[...] = o[:, :R * D]
    self_ref[...] = o[:, R * D:]


def _mm_mid_body(acc_ref, sl_ref, w_ref, hr_ref, self_ref):
    a = acc_ref[...]
    h = jax.nn.relu(a[0] + a[1] + sl_ref[...])
    o = jnp.dot(h.astype(jnp.bfloat16), w_ref[...],
                preferred_element_type=jnp.float32)
    hr_ref[...] = o[:, :R * D]
    self_ref[...] = o[:, R * D:]


def _combine_body(acc_ref, sl_ref, out_ref):
    a = acc_ref[...]
    out_ref[...] = a[0] + a[1] + sl_ref[...]


def _mm_first(x, wcat):
    return pl.pallas_call(
        _mm_first_body,
        grid=(N // _BN,),
        in_specs=[
            pl.BlockSpec((_BN, D), lambda i: (i, 0)),
            pl.BlockSpec((D, R * D + D), lambda i: (0, 0)),
        ],
        out_specs=[
            pl.BlockSpec((_BN, R * D), lambda i: (i, 0)),
            pl.BlockSpec((_BN, D), lambda i: (i, 0)),
        ],
        out_shape=[
            jax.ShapeDtypeStruct((N, R * D), jnp.float32),
            jax.ShapeDtypeStruct((N, D), jnp.float32),
        ],
    )(x, wcat)


def _mm_mid(acc, sl, wcat):
    return pl.pallas_call(
        _mm_mid_body,
        grid=(N // _BN,),
        in_specs=[
            pl.BlockSpec((NC, _BN, D), lambda i: (0, i, 0)),
            pl.BlockSpec((_BN, D), lambda i: (i, 0)),
            pl.BlockSpec((D, R * D + D), lambda i: (0, 0)),
        ],
        out_specs=[
            pl.BlockSpec((_BN, R * D), lambda i: (i, 0)),
            pl.BlockSpec((_BN, D), lambda i: (i, 0)),
        ],
        out_shape=[
            jax.ShapeDtypeStruct((N, R * D), jnp.float32),
            jax.ShapeDtypeStruct((N, D), jnp.float32),
        ],
    )(acc, sl, wcat)


def _combine(acc, sl):
    return pl.pallas_call(
        _combine_body,
        grid=(N // _BN,),
        in_specs=[
            pl.BlockSpec((NC, _BN, D), lambda i: (0, i, 0)),
            pl.BlockSpec((_BN, D), lambda i: (i, 0)),
        ],
        out_specs=pl.BlockSpec((_BN, D), lambda i: (i, 0)),
        out_shape=jax.ShapeDtypeStruct((N, D), jnp.float32),
    )(acc, sl)


def _wcat(a, V, Ws):
    w = jnp.einsum('rb,bio->rio', a, V)
    wc = jnp.concatenate([w.transpose(1, 0, 2).reshape(D, R * D), Ws], axis=1)
    return wc.astype(jnp.bfloat16)


def kernel(x, edge_index, edge_type, V1, a1, Ws1, V2, a2, Ws2):
    src = edge_index[0]
    dst = edge_index[1]
    et = edge_type

    row_idx = src * R + et
    key = dst * R + et

    pad_seq = jax.lax.iota(jnp.int32, EPAD)
    rp = jnp.concatenate([row_idx, pad_seq % (N * R)]).reshape(NW, NBC, 128)
    dp = jnp.concatenate([dst, N + pad_seq % (NPAD - N)]).reshape(NW, NBK, BE)
    kp = jnp.concatenate([key, N * R + pad_seq % (NR_PAD - N * R)]).reshape(NW, NBC, 128)

    norm = _norm_kernel(kp)

    hr1, sl1 = _mm_first(x, _wcat(a1, V1, Ws1))
    acc1 = _agg_kernel(hr1.reshape(N * R, D), rp, dp, norm)
    hr2, sl2 = _mm_mid(acc1, sl1, _wcat(a2, V2, Ws2))
    acc2 = _agg_kernel(hr2.reshape(N * R, D), rp, dp, norm)
    return _combine(acc2, sl2)


# speedup vs baseline: 1.0285x; 1.0285x over previous
import functools

import jax
import jax.numpy as jnp
from jax import lax
from jax.experimental import pallas as pl
from jax.experimental.pallas import tpu as pltpu
from jax.experimental.pallas import tpu_sc as plsc

N = 10000
E = 160000
R = 8
D = 128

NC = 2
NS = 16
NW = NC * NS

EPW = 5120
NBK = EPW // 128
EPAD = NW * EPW - E
NR_PAD = 81920
NPAD = 10240
RPT = NPAD // NS

_mesh = plsc.VectorSubcoreMesh(core_axis_name="c", subcore_axis_name="s")


@functools.partial(
    pl.kernel,
    out_type=jax.ShapeDtypeStruct((NW, EPW), jnp.float32),
    mesh=_mesh,
    compiler_params=pltpu.CompilerParams(needs_layout_passes=False),
    scratch_types=[
        pltpu.VMEM((NBK, 128), jnp.int32),
        pltpu.VMEM((5120,), jnp.float32),
        pltpu.VMEM((EPW,), jnp.float32),
        pltpu.VMEM((128,), jnp.float32),
        pltpu.VMEM((128,), jnp.float32),
        pltpu.VMEM_SHARED((NR_PAD,), jnp.float32),
        pltpu.SemaphoreType.DMA,
    ],
)
def _norm_kernel(key_hbm, norm_hbm, keyv, zbuf, normv, onesv, cbuf, cnt_sp,
                 csem):
    c = lax.axis_index("c")
    s = lax.axis_index("s")
    wid = s * NC + c

    zlen = NR_PAD // NS

    def _z16(i, _):
        zbuf[pl.ds(i * 16, 16)] = jnp.zeros((16,), jnp.float32)
        return 0

    lax.fori_loop(0, zlen // 16, _z16, 0)
    pltpu.sync_copy(zbuf.at[pl.ds(0, zlen)], cnt_sp.at[pl.ds(s * zlen, zlen)])

    def _o16(i, _):
        onesv[pl.ds(i * 16, 16)] = jnp.ones((16,), jnp.float32)
        return 0

    lax.fori_loop(0, 8, _o16, 0)
    plsc.subcore_barrier()

    def _count_row(rr, _):
        row = 2 * s + rr
        pltpu.sync_copy(key_hbm.at[row], keyv)

        def _b(b, _2):
            pltpu.sync_copy(onesv, cnt_sp.at[keyv.at[b]], add=True)
            return 0

        lax.fori_loop(0, NBK, _b, 0)
        return 0

    lax.fori_loop(0, 2, _count_row, 0)
    plsc.subcore_barrier()

    pltpu.sync_copy(key_hbm.at[wid], keyv)

    def _nb(b, _):
        pltpu.async_copy(cnt_sp.at[keyv.at[b]], cbuf, csem).wait()
        for g in range(8):
            c16 = cbuf[pl.ds(g * 16, 16)]
            normv[pl.ds(b * 128 + g * 16, 16)] = 1.0 / jnp.maximum(c16, 1.0)
        return 0

    lax.fori_loop(0, NBK, _nb, 0)
    pltpu.sync_copy(normv, norm_hbm.at[wid])


@functools.partial(
    pl.kernel,
    out_type=jax.ShapeDtypeStruct((NC, NPAD, D), jnp.float32),
    mesh=_mesh,
    compiler_params=pltpu.CompilerParams(needs_layout_passes=False),
    scratch_types=[
        pltpu.VMEM((NBK, 128), jnp.int32),
        pltpu.VMEM((NBK, 128), jnp.int32),
        pltpu.VMEM((EPW,), jnp.float32),
        pltpu.VMEM((128, D), jnp.float32),
        pltpu.VMEM((128, D), jnp.float32),
        pltpu.VMEM_SHARED((NPAD, D), jnp.float32),
        pltpu.SemaphoreType.DMA,
        pltpu.SemaphoreType.DMA,
    ],
)
def _agg_kernel(hr_hbm, idx_hbm, dst_hbm, norm_hbm, out_hbm,
                idxv, dstv, normv, rows0, rows1, acc_sp, gsem0, gsem1):
    c = lax.axis_index("c")
    s = lax.axis_index("s")
    wid = s * NC + c

    pltpu.sync_copy(idx_hbm.at[wid], idxv)
    pltpu.sync_copy(dst_hbm.at[wid], dstv)
    pltpu.sync_copy(norm_hbm.at[wid], normv)

    def _zr(i, _):
        for k in range(D // 16):
            rows0[i, pl.ds(k * 16, 16)] = jnp.zeros((16,), jnp.float32)
        return 0

    lax.fori_loop(0, 128, _zr, 0)
    for t in range(RPT // 128):
        pltpu.sync_copy(rows0, acc_sp.at[pl.ds(s * RPT + t * 128, 128)])
    plsc.subcore_barrier()

    def _scale_block(rows, b):
        def _edge(j, e16):
            nsp = plsc.load_gather(normv, [e16])
            for k in range(D // 16):
                rows[j, pl.ds(k * 16, 16)] = rows[j, pl.ds(k * 16, 16)] * nsp
            return e16 + 1

        lax.fori_loop(0, 128, _edge, lax.broadcast(b * 128, (16,)))

    def _gather_start(b, rows, gsem):
        pltpu.async_copy(hr_hbm.at[idxv.at[b]], rows, gsem)

    def _gather_wait(rows, gsem):
        pltpu.make_async_copy(hr_hbm.at[idxv.at[0]], rows, gsem).wait()

    _gather_start(0, rows0, gsem0)

    def _pair(b2, _):
        b = 2 * b2
        _gather_wait(rows0, gsem0)
        _gather_start(b + 1, rows1, gsem1)
        _scale_block(rows0, b)
        pltpu.sync_copy(rows0, acc_sp.at[dstv.at[b]], add=True)
        _gather_wait(rows1, gsem1)

        @pl.when(b2 < NBK // 2 - 1)
        def _():
            _gather_start(b + 2, rows0, gsem0)

        _scale_block(rows1, b + 1)
        pltpu.sync_copy(rows1, acc_sp.at[dstv.at[b + 1]], add=True)
        return 0

    lax.fori_loop(0, NBK // 2, _pair, 0)
    plsc.subcore_barrier()
    pltpu.sync_copy(acc_sp.at[pl.ds(s * RPT, RPT)],
                    out_hbm.at[c, pl.ds(s * RPT, RPT)])


_BN = 1000


def _mm_first_body(x_ref, w_ref, hr_ref, self_ref):
    o = jnp.dot(x_ref[...].astype(jnp.bfloat16), w_ref[...],
                preferred_element_type=jnp.float32)
    hr_ref[...] = o[:, :R * D]
    self_ref[...] = o[:, R * D:]


def _mm_mid_body(acc_ref, sl_ref, w_ref, hr_ref, self_ref):
    a = acc_ref[...]
    h = jax.nn.relu(a[0] + a[1] + sl_ref[...])
    o = jnp.dot(h.astype(jnp.bfloat16), w_ref[...],
                preferred_element_type=jnp.float32)
    hr_ref[...] = o[:, :R * D]
    self_ref[...] = o[:, R * D:]


def _combine_body(acc_ref, sl_ref, out_ref):
    a = acc_ref[...]
    out_ref[...] = a[0] + a[1] + sl_ref[...]


def _mm_first(x, wcat):
    return pl.pallas_call(
        _mm_first_body,
        grid=(N // _BN,),
        in_specs=[
            pl.BlockSpec((_BN, D), lambda i: (i, 0)),
            pl.BlockSpec((D, R * D + D), lambda i: (0, 0)),
        ],
        out_specs=[
            pl.BlockSpec((_BN, R * D), lambda i: (i, 0)),
            pl.BlockSpec((_BN, D), lambda i: (i, 0)),
        ],
        out_shape=[
            jax.ShapeDtypeStruct((N, R * D), jnp.float32),
            jax.ShapeDtypeStruct((N, D), jnp.float32),
        ],
    )(x, wcat)


def _mm_mid(acc, sl, wcat):
    return pl.pallas_call(
        _mm_mid_body,
        grid=(N // _BN,),
        in_specs=[
            pl.BlockSpec((NC, _BN, D), lambda i: (0, i, 0)),
            pl.BlockSpec((_BN, D), lambda i: (i, 0)),
            pl.BlockSpec((D, R * D + D), lambda i: (0, 0)),
        ],
        out_specs=[
            pl.BlockSpec((_BN, R * D), lambda i: (i, 0)),
            pl.BlockSpec((_BN, D), lambda i: (i, 0)),
        ],
        out_shape=[
            jax.ShapeDtypeStruct((N, R * D), jnp.float32),
            jax.ShapeDtypeStruct((N, D), jnp.float32),
        ],
    )(acc, sl, wcat)


def _combine(acc, sl):
    return pl.pallas_call(
        _combine_body,
        grid=(N // _BN,),
        in_specs=[
            pl.BlockSpec((NC, _BN, D), lambda i: (0, i, 0)),
            pl.BlockSpec((_BN, D), lambda i: (i, 0)),
        ],
        out_specs=pl.BlockSpec((_BN, D), lambda i: (i, 0)),
        out_shape=jax.ShapeDtypeStruct((N, D), jnp.float32),
    )(acc, sl)


def _wcat(a, V, Ws):
    w = jnp.einsum('rb,bio->rio', a, V)
    wc = jnp.concatenate([w.transpose(1, 0, 2).reshape(D, R * D), Ws], axis=1)
    return wc.astype(jnp.bfloat16)


def kernel(x, edge_index, edge_type, V1, a1, Ws1, V2, a2, Ws2):
    src = edge_index[0]
    dst = edge_index[1]
    et = edge_type

    row_idx = src * R + et
    key = dst * R + et

    pad_seq = jax.lax.iota(jnp.int32, EPAD)
    rp = jnp.concatenate([row_idx, pad_seq % (N * R)]).reshape(NW, NBK, 128)
    dp = jnp.concatenate([dst, N + pad_seq % (NPAD - N)]).reshape(NW, NBK, 128)
    kp = jnp.concatenate([key, N * R + pad_seq % (NR_PAD - N * R)]).reshape(NW, NBK, 128)

    norm = _norm_kernel(kp)

    hr1, sl1 = _mm_first(x, _wcat(a1, V1, Ws1))
    acc1 = _agg_kernel(hr1.reshape(N * R, D), rp, dp, norm)
    hr2, sl2 = _mm_mid(acc1, sl1, _wcat(a2, V2, Ws2))
    acc2 = _agg_kernel(hr2.reshape(N * R, D), rp, dp, norm)
    return _combine(acc2, sl2)
